# baseline (speedup 1.0000x reference)
import jax
import jax.numpy as jnp
from jax import lax
from jax.experimental import pallas as pl
from jax.experimental.pallas import tpu as pltpu

N = 4096
F = 128
C = 64
B = 1024
K = 64
JT = 512
MT = 512
RT = 256
XT = 1024
AT = B // (N // XT)

_F32_MAX = float(jnp.finfo(jnp.float32).max)


def _bf(x):
    return x.astype(jnp.bfloat16)


def _dot(a, b):
    return jnp.dot(_bf(a), _bf(b), preferred_element_type=jnp.float32)


def _mlp_body(x_ref, w1_ref, w2_ref, w3_ref, a_ref, lbf_ref, lf_ref, ab_ref):
    h = jax.nn.relu(_dot(x_ref[...], w1_ref[...]))
    h = jax.nn.relu(_dot(h, w2_ref[...]))
    logits = _dot(h, w3_ref[...])
    lf_ref[...] = logits
    lbf_ref[...] = _bf(logits)
    ab_ref[...] = _bf(a_ref[...] + 1e-12)


def _run_mlp(X, W1, W2, W3, ppr):
    grid = (N // XT,)
    return pl.pallas_call(
        _mlp_body,
        grid=grid,
        in_specs=[
            pl.BlockSpec((XT, F), lambda i: (i, 0)),
            pl.BlockSpec((F, 512), lambda i: (0, 0)),
            pl.BlockSpec((512, 256), lambda i: (0, 0)),
            pl.BlockSpec((256, C), lambda i: (0, 0)),
            pl.BlockSpec((AT, N), lambda i: (i, 0)),
        ],
        out_specs=(
            pl.BlockSpec((XT, C), lambda i: (i, 0)),
            pl.BlockSpec((XT, C), lambda i: (i, 0)),
            pl.BlockSpec((AT, N), lambda i: (i, 0)),
        ),
        out_shape=(
            jax.ShapeDtypeStruct((N, C), jnp.bfloat16),
            jax.ShapeDtypeStruct((N, C), jnp.float32),
            jax.ShapeDtypeStruct((B, N), jnp.bfloat16),
        ),
    )(X, W1, W2, W3, ppr)


def _wd_body(ab_ref, lbf_ref, lf_ref, out_ref):
    m = pl.program_id(0)
    lm_bf = lbf_ref[pl.ds(m * MT, MT), :]
    lm_f = lf_ref[pl.ds(m * MT, MT), :]
    lm_sq = lm_f * lm_f
    ones_1c = jnp.ones((1, C), jnp.float32)
    sqm_row = jax.lax.dot_general(
        ones_1c, lm_sq, (((1,), (1,)), ((), ())),
        preferred_element_type=jnp.float32,
        precision=jax.lax.Precision.HIGHEST)

    acc = jnp.zeros((B, MT), jnp.float32)
    for j in range(N // JT):
        lj_bf = lbf_ref[pl.ds(j * JT, JT), :]
        lj_f = lf_ref[pl.ds(j * JT, JT), :]
        sqj_col = jnp.sum(lj_f * lj_f, axis=1, keepdims=True)
        g = jnp.dot(lj_bf, lm_bf.T, preferred_element_type=jnp.float32)
        d2 = (sqj_col + sqm_row) - 2.0 * g
        l2 = jnp.sqrt(jnp.maximum(d2, 1e-12))
        acc = acc + jnp.dot(ab_ref[:, pl.ds(j * JT, JT)], _bf(l2),
                            preferred_element_type=jnp.float32)
    out_ref[...] = acc


def _run_wd(ab, logits_bf, logits_f):
    grid = (N // MT,)
    return pl.pallas_call(
        _wd_body,
        grid=grid,
        in_specs=[
            pl.BlockSpec((B, N), lambda m: (0, 0)),
            pl.BlockSpec((N, C), lambda m: (0, 0)),
            pl.BlockSpec((N, C), lambda m: (0, 0)),
        ],
        out_specs=pl.BlockSpec((B, MT), lambda m: (0, m)),
        out_shape=jax.ShapeDtypeStruct((B, N), jnp.float32),
    )(ab, logits_bf, logits_f)


def _sel_body(a_ref, wd_ref, lbf_ref, out_ref):
    a2 = a_ref[...] + 1e-12
    bits = jax.lax.bitcast_convert_type(a2, jnp.int32)
    kf = jnp.float32(K)
    ones_n1 = jnp.ones((N, 1), jnp.bfloat16)

    def _cnt(mask):
        mbf = _bf(jnp.where(mask, 1.0, 0.0))
        return jnp.dot(mbf, ones_n1, preferred_element_type=jnp.float32)

    def _cnt_ge_bits(t):
        return _cnt(a2 >= jax.lax.bitcast_convert_type(t, jnp.float32))

    def _vcond(carry):
        lo, hi = carry
        return jnp.max(hi - lo) > 1

    def _vstep(carry):
        lo, hi = carry
        mid = lo + ((hi - lo) >> 1)
        c = _cnt_ge_bits(mid)
        ge = c >= kf
        exact = c == kf
        lo = jnp.where(exact, mid, jnp.where(ge, mid, lo))
        hi = jnp.where(exact, mid + 1, jnp.where(ge, hi, mid))
        return lo, hi

    lo0 = jnp.zeros((RT, 1), jnp.int32)
    hi0 = jnp.full((RT, 1), jnp.int32(0x7F800000))
    vstar, _ = lax.while_loop(_vcond, _vstep, (lo0, hi0))
    n_gt = _cnt_ge_bits(vstar + 1)
    r = kf - n_gt
    eq = bits == vstar
    n_eq = _cnt(eq)
    iota = lax.broadcasted_iota(jnp.int32, (RT, N), 1)

    def _icond(carry):
        lo, hi = carry
        return jnp.max(hi - lo) > 1

    def _istep(carry):
        lo, hi = carry
        mid = lo + ((hi - lo) >> 1)
        ge = _cnt(eq & (iota <= mid)) >= r
        return jnp.where(ge, lo, mid), jnp.where(ge, mid, hi)

    take_all = r >= n_eq
    ilo0 = jnp.where(take_all, jnp.int32(N - 2), jnp.int32(-1))
    ihi0 = jnp.full((RT, 1), jnp.int32(N - 1))
    _, cstar = lax.while_loop(_icond, _istep, (ilo0, ihi0))

    sel = (bits > vstar) | (eq & (iota <= cstar))

    d = jnp.where(a2 > 0.0, wd_ref[...], _F32_MAX)
    neg_d = jnp.where(sel, -d, -_F32_MAX)
    xmax = jnp.max(neg_d, axis=1, keepdims=True)
    e = jnp.where(sel, jnp.exp(neg_d - xmax), 0.0)
    s1 = jnp.sum(e, axis=1, keepdims=True)
    soft = e / s1
    soft = soft * a2
    soft = jnp.where(sel, soft, 0.0)
    soft = soft / jnp.sum(soft, axis=1, keepdims=True)
    row_sum = jnp.sum(a2, axis=1, keepdims=True)
    agg = jnp.dot(_bf(soft), lbf_ref[...], preferred_element_type=jnp.float32)
    out_ref[...] = row_sum * agg


def _run_sel(ppr, wdt, logits_bf):
    grid = (B // RT,)
    return pl.pallas_call(
        _sel_body,
        grid=grid,
        in_specs=[
            pl.BlockSpec((RT, N), lambda r: (r, 0)),
            pl.BlockSpec((RT, N), lambda r: (r, 0)),
            pl.BlockSpec((N, C), lambda r: (0, 0)),
        ],
        out_specs=pl.BlockSpec((RT, C), lambda r: (r, 0)),
        out_shape=jax.ShapeDtypeStruct((B, C), jnp.float32),
    )(ppr, wdt, logits_bf)


@jax.jit
def kernel(X, ppr_scores, W1, W2, W3):
    logits_bf, logits_f, ab = _run_mlp(X, W1, W2, W3, ppr_scores)
    wdt = _run_wd(ab, logits_bf, logits_f)
    return _run_sel(ppr_scores, wdt, logits_bf)


# speedup vs baseline: 1.1509x; 1.1509x over previous
import numpy as np

import jax
import jax.numpy as jnp
from jax import lax
from jax.experimental import pallas as pl
from jax.experimental.pallas import tpu as pltpu

N = 4096
F = 128
C = 64
B = 1024
K = 64
JT = 512
MT = 512
RT = 256
XT = 1024
AT = B // (N // XT)

_F32_MAX = float(jnp.finfo(jnp.float32).max)

_LO0 = int(np.float32(1e-12).view(np.int32))
_HI0 = int(np.float32(1.0).view(np.int32))
_VITERS = int(np.ceil(np.log2(float(_HI0 - _LO0))))
_MSTEPS = N // MT
_IT_PER_STEP = -(-_VITERS // _MSTEPS)


def _bf(x):
    return x.astype(jnp.bfloat16)


def _dot(a, b):
    return jnp.dot(_bf(a), _bf(b), preferred_element_type=jnp.float32)


def _mlp_body(x_ref, w1_ref, w2_ref, w3_ref, p_ref, lbf_ref, lf_ref,
              a2_ref, ab_ref):
    h = jax.nn.relu(_dot(x_ref[...], w1_ref[...]))
    h = jax.nn.relu(_dot(h, w2_ref[...]))
    logits = _dot(h, w3_ref[...])
    lf_ref[...] = logits
    lbf_ref[...] = _bf(logits)
    a2 = p_ref[...] + 1e-12
    a2_ref[...] = a2
    ab_ref[...] = _bf(a2)


def _run_mlp(X, W1, W2, W3, ppr):
    grid = (N // XT,)
    return pl.pallas_call(
        _mlp_body,
        grid=grid,
        in_specs=[
            pl.BlockSpec((XT, F), lambda i: (i, 0)),
            pl.BlockSpec((F, 512), lambda i: (0, 0)),
            pl.BlockSpec((512, 256), lambda i: (0, 0)),
            pl.BlockSpec((256, C), lambda i: (0, 0)),
            pl.BlockSpec((AT, N), lambda i: (i, 0)),
        ],
        out_specs=(
            pl.BlockSpec((XT, C), lambda i: (i, 0)),
            pl.BlockSpec((XT, C), lambda i: (i, 0)),
            pl.BlockSpec((AT, N), lambda i: (i, 0)),
            pl.BlockSpec((AT, N), lambda i: (i, 0)),
        ),
        out_shape=(
            jax.ShapeDtypeStruct((N, C), jnp.bfloat16),
            jax.ShapeDtypeStruct((N, C), jnp.float32),
            jax.ShapeDtypeStruct((B, N), jnp.float32),
            jax.ShapeDtypeStruct((B, N), jnp.bfloat16),
        ),
    )(X, W1, W2, W3, ppr)


def _wd_body(a2_ref, ab_ref, lbf_ref, lf_ref, out_ref, vstar_ref, cstar_ref,
             lo_ref, hi_ref):
    m = pl.program_id(0)

    @pl.when(m == 0)
    def _init():
        lo_ref[...] = jnp.full((B, 1), jnp.int32(_LO0))
        hi_ref[...] = jnp.full((B, 1), jnp.int32(_HI0))

    lm_bf = lbf_ref[pl.ds(m * MT, MT), :]
    lm_f = lf_ref[pl.ds(m * MT, MT), :]
    lm_sq = lm_f * lm_f
    ones_1c = jnp.ones((1, C), jnp.float32)
    sqm_row = jax.lax.dot_general(
        ones_1c, lm_sq, (((1,), (1,)), ((), ())),
        preferred_element_type=jnp.float32,
        precision=jax.lax.Precision.HIGHEST)

    a2 = a2_ref[...]
    kf = jnp.float32(K)

    def _cnt_ge(tbits):
        t = jax.lax.bitcast_convert_type(tbits, jnp.float32)
        return jnp.sum(jnp.where(a2 >= t, 1.0, 0.0), axis=1, keepdims=True)

    lo = lo_ref[...]
    hi = hi_ref[...]
    acc = jnp.zeros((B, MT), jnp.float32)
    for j in range(N // JT):
        lj_bf = lbf_ref[pl.ds(j * JT, JT), :]
        lj_f = lf_ref[pl.ds(j * JT, JT), :]
        sqj_col = jnp.sum(lj_f * lj_f, axis=1, keepdims=True)
        g = jnp.dot(lj_bf, lm_bf.T, preferred_element_type=jnp.float32)
        d2 = (sqj_col + sqm_row) - 2.0 * g
        l2 = jnp.sqrt(jnp.maximum(d2, 1e-12))
        acc = acc + jnp.dot(ab_ref[:, pl.ds(j * JT, JT)], _bf(l2),
                            preferred_element_type=jnp.float32)
        if j % 2 == 1:
            mid = lo + ((hi - lo) >> 1)
            ge = _cnt_ge(mid) >= kf
            lo = jnp.where(ge, mid, lo)
            hi = jnp.where(ge, hi, mid)
    out_ref[...] = acc
    lo_ref[...] = lo
    hi_ref[...] = hi

    @pl.when(m == _MSTEPS - 1)
    def _finish():
        vstar = lo_ref[...]
        vf = jax.lax.bitcast_convert_type(vstar, jnp.float32)
        vf1 = jax.lax.bitcast_convert_type(vstar + 1, jnp.float32)
        n_gt = jnp.sum(jnp.where(a2 >= vf1, 1.0, 0.0), axis=1, keepdims=True)
        eq = a2 == vf
        n_eq = jnp.sum(jnp.where(eq, 1.0, 0.0), axis=1, keepdims=True)
        r = kf - n_gt
        iota = lax.broadcasted_iota(jnp.int32, (B, N), 1)

        def _icond(carry):
            ilo, ihi = carry
            return jnp.max(ihi - ilo) > 1

        def _istep(carry):
            ilo, ihi = carry
            mid = ilo + ((ihi - ilo) >> 1)
            cnt = jnp.sum(jnp.where(eq & (iota <= mid), 1.0, 0.0), axis=1,
                          keepdims=True)
            ge = cnt >= r
            return jnp.where(ge, ilo, mid), jnp.where(ge, mid, ihi)

        take_all = r >= n_eq
        ilo0 = jnp.where(take_all, jnp.int32(N - 2), jnp.int32(-1))
        ihi0 = jnp.full((B, 1), jnp.int32(N - 1))
        _, cstar = lax.while_loop(_icond, _istep, (ilo0, ihi0))
        vstar_ref[...] = vstar
        cstar_ref[...] = cstar


def _run_wd(a2, ab, logits_bf, logits_f):
    grid = (_MSTEPS,)
    return pl.pallas_call(
        _wd_body,
        grid=grid,
        in_specs=[
            pl.BlockSpec((B, N), lambda m: (0, 0)),
            pl.BlockSpec((B, N), lambda m: (0, 0)),
            pl.BlockSpec((N, C), lambda m: (0, 0)),
            pl.BlockSpec((N, C), lambda m: (0, 0)),
        ],
        out_specs=(
            pl.BlockSpec((B, MT), lambda m: (0, m)),
            pl.BlockSpec((B, 1), lambda m: (0, 0)),
            pl.BlockSpec((B, 1), lambda m: (0, 0)),
        ),
        out_shape=(
            jax.ShapeDtypeStruct((B, N), jnp.float32),
            jax.ShapeDtypeStruct((B, 1), jnp.int32),
            jax.ShapeDtypeStruct((B, 1), jnp.int32),
        ),
        scratch_shapes=[
            pltpu.VMEM((B, 1), jnp.int32),
            pltpu.VMEM((B, 1), jnp.int32),
        ],
    )(a2, ab, logits_bf, logits_f)


def _sel_body(a2_ref, wd_ref, vstar_ref, cstar_ref, lbf_ref, out_ref):
    a2 = a2_ref[...]
    vstar = vstar_ref[...]
    vf = jax.lax.bitcast_convert_type(vstar, jnp.float32)
    iota = lax.broadcasted_iota(jnp.int32, (RT, N), 1)
    sel = (a2 > vf) | ((a2 == vf) & (iota <= cstar_ref[...]))

    d = jnp.where(a2 > 0.0, wd_ref[...], _F32_MAX)
    neg_d = jnp.where(sel, -d, -_F32_MAX)
    xmax = jnp.max(neg_d, axis=1, keepdims=True)
    e = jnp.where(sel, jnp.exp(neg_d - xmax), 0.0)
    s1 = jnp.sum(e, axis=1, keepdims=True)
    soft = e / s1
    soft = soft * a2
    soft = jnp.where(sel, soft, 0.0)
    soft = soft / jnp.sum(soft, axis=1, keepdims=True)
    row_sum = jnp.sum(a2, axis=1, keepdims=True)
    agg = jnp.dot(_bf(soft), lbf_ref[...], preferred_element_type=jnp.float32)
    out_ref[...] = row_sum * agg


def _run_sel(a2, wdt, vstar, cstar, logits_bf):
    grid = (B // RT,)
    return pl.pallas_call(
        _sel_body,
        grid=grid,
        in_specs=[
            pl.BlockSpec((RT, N), lambda r: (r, 0)),
            pl.BlockSpec((RT, N), lambda r: (r, 0)),
            pl.BlockSpec((RT, 1), lambda r: (r, 0)),
            pl.BlockSpec((RT, 1), lambda r: (r, 0)),
            pl.BlockSpec((N, C), lambda r: (0, 0)),
        ],
        out_specs=pl.BlockSpec((RT, C), lambda r: (r, 0)),
        out_shape=jax.ShapeDtypeStruct((B, C), jnp.float32),
    )(a2, wdt, vstar, cstar, logits_bf)


@jax.jit
def kernel(X, ppr_scores, W1, W2, W3):
    logits_bf, logits_f, a2, ab = _run_mlp(X, W1, W2, W3, ppr_scores)
    wdt, vstar, cstar = _run_wd(a2, ab, logits_bf, logits_f)
    return _run_sel(a2, wdt, vstar, cstar, logits_bf)
